# baseline (speedup 1.0000x reference)
import functools

import jax
import jax.numpy as jnp
from jax.experimental import pallas as pl
from jax.experimental.pallas import tpu as pltpu
from jax.experimental.pallas import tpu_sc as plsc

_SC_CORES = 2
_SC_SUBCORES = 16
_SC_WORKERS = _SC_CORES * _SC_SUBCORES
_IDX_CHUNK = 80
_PACK = 4


def _sc_gather(tbl4, indices):
    nw, nch, ch = indices.shape
    per_w = nch * ch
    n = nw * per_w
    d = tbl4.shape[1]
    mesh = plsc.VectorSubcoreMesh(core_axis_name="c", subcore_axis_name="s")

    @pl.kernel(
        out_type=jax.ShapeDtypeStruct((n, d), tbl4.dtype),
        mesh=mesh,
        scratch_types=[
            pltpu.VMEM((nch, ch), jnp.int32),
            pltpu.VMEM((per_w, d), tbl4.dtype),
            pltpu.SemaphoreType.DMA,
        ],
    )
    def gather_kernel(tbl_hbm, idx_hbm, out_hbm, idx_v, rows_v, sem):
        wid = jax.lax.axis_index("s") * _SC_CORES + jax.lax.axis_index("c")
        pltpu.sync_copy(idx_hbm.at[wid], idx_v)
        copies = [
            pltpu.async_copy(
                tbl_hbm.at[idx_v.at[c]], rows_v.at[pl.ds(c * ch, ch)], sem
            )
            for c in range(nch)
        ]
        for cp in copies:
            cp.wait()
        pltpu.sync_copy(rows_v, out_hbm.at[pl.ds(wid * per_w, per_w)])

    return gather_kernel(tbl4, indices)


def _pool_body(g_ref, o_ref, mean_ref, *, batch, ctx, d):
    lane_group = jax.lax.broadcasted_iota(jnp.int32, (batch, _PACK * d), 1) // d
    acc = jnp.zeros((batch, _PACK * d), jnp.float32)
    for j in range(ctx):
        g_slab = g_ref[pl.ds(j * batch, batch), :]
        o_slab = o_ref[pl.ds(j * batch, batch), :]
        acc = acc + jnp.where(lane_group == o_slab, g_slab, 0.0)
    folded = acc[:, 0:d]
    for g in range(1, _PACK):
        folded = folded + acc[:, g * d:(g + 1) * d]
    mean_ref[...] = (folded * (1.0 / ctx)).astype(jnp.bfloat16)


def _pool(gathered, offs, batch, ctx, interpret=False):
    n, dp = gathered.shape
    d = dp // _PACK
    return pl.pallas_call(
        functools.partial(_pool_body, batch=batch, ctx=ctx, d=d),
        in_specs=[
            pl.BlockSpec((n, dp), lambda: (0, 0)),
            pl.BlockSpec((n, 1), lambda: (0, 0)),
        ],
        out_specs=pl.BlockSpec((batch, d), lambda: (0, 0)),
        out_shape=jax.ShapeDtypeStruct((batch, d), jnp.bfloat16),
        interpret=interpret,
    )(gathered, offs)


def _mm_body(m_ref, w_ref, o_ref):
    o_ref[...] = jax.lax.dot_general(
        m_ref[...],
        w_ref[...].astype(jnp.bfloat16),
        (((1,), (1,)), ((), ())),
        preferred_element_type=jnp.float32,
    )


def _project(mean_bf, w_out, v_tile=2048, interpret=False):
    batch, d = mean_bf.shape
    vocab = w_out.shape[0]
    grid = (pl.cdiv(vocab, v_tile),)
    return pl.pallas_call(
        _mm_body,
        grid=grid,
        in_specs=[
            pl.BlockSpec((batch, d), lambda i: (0, 0)),
            pl.BlockSpec((v_tile, d), lambda i: (i, 0)),
        ],
        out_specs=pl.BlockSpec((batch, v_tile), lambda i: (0, i)),
        out_shape=jax.ShapeDtypeStruct((batch, vocab), jnp.float32),
        compiler_params=pltpu.CompilerParams(dimension_semantics=("parallel",)),
        interpret=interpret,
    )(mean_bf, w_out)


def kernel(contexts, emb_table, W_out):
    batch, ctx = contexts.shape
    vocab, d = emb_table.shape
    n = batch * ctx
    idx = contexts.T.reshape(n).astype(jnp.int32)
    tbl4 = emb_table.reshape(vocab // _PACK, _PACK * d)
    q = (idx // _PACK).reshape(
        _SC_WORKERS, n // (_SC_WORKERS * _IDX_CHUNK), _IDX_CHUNK
    )
    offs = (idx % _PACK).reshape(n, 1)
    gathered = _sc_gather(tbl4, q)
    mean_bf = _pool(gathered, offs, batch, ctx)
    return _project(mean_bf, W_out)


# speedup vs baseline: 2.4469x; 2.4469x over previous
import functools

import jax
import jax.numpy as jnp
from jax.experimental import pallas as pl
from jax.experimental.pallas import tpu as pltpu
from jax.experimental.pallas import tpu_sc as plsc

_SC_CORES = 2
_SC_SUBCORES = 16
_SC_WORKERS = _SC_CORES * _SC_SUBCORES
_IDX_CHUNK = 80
_PACK = 4


def _sc_gather(tbl4, indices):
    nw, nch, ch = indices.shape
    per_w = nch * ch
    n = nw * per_w
    d = tbl4.shape[1]
    mesh = plsc.VectorSubcoreMesh(core_axis_name="c", subcore_axis_name="s")

    @pl.kernel(
        out_type=jax.ShapeDtypeStruct((n, d), tbl4.dtype),
        mesh=mesh,
        scratch_types=[
            pltpu.VMEM((nch, ch), jnp.int32),
            pltpu.VMEM((per_w, d), tbl4.dtype),
            pltpu.SemaphoreType.DMA,
        ],
    )
    def gather_kernel(tbl_hbm, idx_hbm, out_hbm, idx_v, rows_v, sem):
        wid = jax.lax.axis_index("s") * _SC_CORES + jax.lax.axis_index("c")
        pltpu.sync_copy(idx_hbm.at[wid], idx_v)
        copies = [
            pltpu.async_copy(
                tbl_hbm.at[idx_v.at[c]], rows_v.at[pl.ds(c * ch, ch)], sem
            )
            for c in range(nch)
        ]
        for cp in copies:
            cp.wait()
        pltpu.sync_copy(rows_v, out_hbm.at[pl.ds(wid * per_w, per_w)])

    return gather_kernel(tbl4, indices)


def _pool_body(g_ref, o_ref, mean_ref, *, batch, ctx, d):
    lane_group = jax.lax.broadcasted_iota(jnp.int32, (batch, _PACK * d), 1) // d
    acc = jnp.zeros((batch, _PACK * d), jnp.float32)
    for j in range(ctx):
        g_slab = g_ref[pl.ds(j * batch, batch), :]
        o_slab = o_ref[pl.ds(j * batch, batch), :]
        acc = acc + jnp.where(lane_group == o_slab, g_slab, 0.0)
    folded = acc[:, 0:d]
    for g in range(1, _PACK):
        folded = folded + acc[:, g * d:(g + 1) * d]
    mean_ref[...] = (folded * (1.0 / ctx)).astype(jnp.bfloat16)


def _pool(gathered, offs, batch, ctx, interpret=False):
    n, dp = gathered.shape
    d = dp // _PACK
    return pl.pallas_call(
        functools.partial(_pool_body, batch=batch, ctx=ctx, d=d),
        in_specs=[
            pl.BlockSpec((n, dp), lambda: (0, 0)),
            pl.BlockSpec((n, 1), lambda: (0, 0)),
        ],
        out_specs=pl.BlockSpec((batch, d), lambda: (0, 0)),
        out_shape=jax.ShapeDtypeStruct((batch, d), jnp.bfloat16),
        interpret=interpret,
    )(gathered, offs)


def _mm_body(m_ref, w_ref, o_ref):
    o_ref[...] = jax.lax.dot_general(
        w_ref[...].astype(jnp.bfloat16),
        m_ref[...],
        (((1,), (1,)), ((), ())),
        preferred_element_type=jnp.float32,
    )


def _project_t(mean_bf, w_out, v_tile=2048, interpret=False):
    batch, d = mean_bf.shape
    vocab = w_out.shape[0]
    grid = (pl.cdiv(vocab, v_tile),)
    return pl.pallas_call(
        _mm_body,
        grid=grid,
        in_specs=[
            pl.BlockSpec((batch, d), lambda i: (0, 0)),
            pl.BlockSpec((v_tile, d), lambda i: (i, 0)),
        ],
        out_specs=pl.BlockSpec((v_tile, batch), lambda i: (i, 0)),
        out_shape=jax.ShapeDtypeStruct((vocab, batch), jnp.float32),
        compiler_params=pltpu.CompilerParams(dimension_semantics=("parallel",)),
        interpret=interpret,
    )(mean_bf, w_out)


def kernel(contexts, emb_table, W_out):
    batch, ctx = contexts.shape
    vocab, d = emb_table.shape
    n = batch * ctx
    idx = contexts.T.reshape(n).astype(jnp.int32)
    tbl4 = emb_table.reshape(vocab // _PACK, _PACK * d)
    q = (idx // _PACK).reshape(
        _SC_WORKERS, n // (_SC_WORKERS * _IDX_CHUNK), _IDX_CHUNK
    )
    offs = (idx % _PACK).reshape(n, 1)
    gathered = _sc_gather(tbl4, q)
    mean_bf = _pool(gathered, offs, batch, ctx)
    return _project_t(mean_bf, W_out).T


# speedup vs baseline: 3.3362x; 1.3635x over previous
import functools

import jax
import jax.numpy as jnp
from jax.experimental import pallas as pl
from jax.experimental.pallas import tpu as pltpu
from jax.experimental.pallas import tpu_sc as plsc

_SC_CORES = 2
_SC_SUBCORES = 16
_SC_WORKERS = _SC_CORES * _SC_SUBCORES


def _sc_gather_elems(flat_tbl, gidx):
    nw, n = gidx.shape
    mesh = plsc.VectorSubcoreMesh(core_axis_name="c", subcore_axis_name="s")

    @pl.kernel(
        out_type=jax.ShapeDtypeStruct((nw, n), flat_tbl.dtype),
        mesh=mesh,
        scratch_types=[
            pltpu.VMEM((n,), jnp.int32),
            pltpu.VMEM((n,), flat_tbl.dtype),
            pltpu.SemaphoreType.DMA,
        ],
    )
    def gather_kernel(tbl_hbm, idx_hbm, out_hbm, idx_v, vals_v, sem):
        wid = jax.lax.axis_index("s") * _SC_CORES + jax.lax.axis_index("c")
        pltpu.sync_copy(idx_hbm.at[wid], idx_v)
        pltpu.async_copy(tbl_hbm.at[idx_v], vals_v, sem).wait()
        pltpu.sync_copy(vals_v, out_hbm.at[wid])

    return gather_kernel(flat_tbl, gidx)


def _pool_t_body(g_ref, mean_ref, *, batch, ctx):
    acc = g_ref[:, pl.ds(0, batch)]
    for j in range(1, ctx):
        acc = acc + g_ref[:, pl.ds(j * batch, batch)]
    mean_ref[...] = (acc * (1.0 / ctx)).astype(jnp.bfloat16)


def _pool_t(g_t, batch, ctx, interpret=False):
    d, n = g_t.shape
    return pl.pallas_call(
        functools.partial(_pool_t_body, batch=batch, ctx=ctx),
        in_specs=[pl.BlockSpec((d, n), lambda: (0, 0))],
        out_specs=pl.BlockSpec((d, batch), lambda: (0, 0)),
        out_shape=jax.ShapeDtypeStruct((d, batch), jnp.bfloat16),
        interpret=interpret,
    )(g_t)


def _mm_t_body(m_ref, w_ref, o_ref):
    o_ref[...] = jax.lax.dot_general(
        w_ref[...],
        m_ref[...],
        (((0,), (0,)), ((), ())),
        preferred_element_type=jnp.float32,
    )


def _project_t(mean_t, w_t, v_tile=2048, interpret=False):
    d, batch = mean_t.shape
    vocab = w_t.shape[1]
    grid = (pl.cdiv(vocab, v_tile),)
    return pl.pallas_call(
        _mm_t_body,
        grid=grid,
        in_specs=[
            pl.BlockSpec((d, batch), lambda i: (0, 0)),
            pl.BlockSpec((d, v_tile), lambda i: (0, i)),
        ],
        out_specs=pl.BlockSpec((v_tile, batch), lambda i: (i, 0)),
        out_shape=jax.ShapeDtypeStruct((vocab, batch), jnp.float32),
        compiler_params=pltpu.CompilerParams(dimension_semantics=("parallel",)),
        interpret=interpret,
    )(mean_t, w_t)


def kernel(contexts, emb_table, W_out):
    batch, ctx = contexts.shape
    vocab, d = emb_table.shape
    n = batch * ctx
    idx = contexts.T.reshape(n).astype(jnp.int32)
    flat_e = emb_table.T.reshape(d * vocab)
    gidx = jnp.arange(d, dtype=jnp.int32)[:, None] * vocab + idx[None, :]
    g_t = _sc_gather_elems(flat_e, gidx)
    mean_t = _pool_t(g_t, batch, ctx)
    w_t = W_out.astype(jnp.bfloat16).T
    return _project_t(mean_t, w_t).T
